# baseline (speedup 1.0000x reference)
import functools

import jax
import jax.numpy as jnp
from jax import lax
from jax.experimental import pallas as pl
from jax.experimental.pallas import tpu as pltpu
from jax.experimental.pallas import tpu_sc as plsc

NUM_CORES = 2
NUM_SUBCORES = 16
NW = NUM_CORES * NUM_SUBCORES
CHUNK = 32
NBUF = 3


@functools.partial(jax.jit, static_argnums=(1, 2))
def _posemb_sc(table, b, t):
    d = table.shape[1]
    rows_per_w = t // NW
    chunk = min(rows_per_w, CHUNK)
    n_chunks = rows_per_w // chunk
    nbuf = min(NBUF, n_chunks)

    mesh = plsc.VectorSubcoreMesh(core_axis_name="c", subcore_axis_name="s")

    scratch = [pltpu.VMEM((chunk, d), jnp.float32) for _ in range(nbuf)]
    scratch += [pltpu.SemaphoreType.DMA for _ in range(2 * nbuf)]

    @functools.partial(
        pl.kernel,
        mesh=mesh,
        out_type=jax.ShapeDtypeStruct((b * t, d), jnp.float32),
        scratch_types=scratch,
    )
    def body(table_hbm, out_hbm, *scr):
        bufs = scr[:nbuf]
        rsems = scr[nbuf:2 * nbuf]
        wsems = scr[2 * nbuf:]
        wid = lax.axis_index("s") * NUM_CORES + lax.axis_index("c")
        base = wid * rows_per_w

        def start_read(c):
            r0 = base + c * chunk
            return pltpu.async_copy(
                table_hbm.at[pl.ds(r0, chunk)], bufs[c % nbuf], rsems[c % nbuf])

        def start_writes(c):
            r0 = base + c * chunk
            return [
                pltpu.async_copy(
                    bufs[c % nbuf], out_hbm.at[pl.ds(bi * t + r0, chunk)],
                    wsems[c % nbuf])
                for bi in range(b)
            ]

        rd = {}
        wr = {}
        for c in range(nbuf):
            rd[c] = start_read(c)
        for c in range(n_chunks):
            rd[c].wait()
            wr[c] = start_writes(c)
            if c + nbuf < n_chunks:
                for w in wr[c]:
                    w.wait()
                rd[c + nbuf] = start_read(c + nbuf)
        for c in range(max(0, n_chunks - nbuf), n_chunks):
            for w in wr[c]:
                w.wait()

    return body(table)


def kernel(x, positional_emb):
    b, t = x.shape
    assert t % NW == 0
    out = _posemb_sc(positional_emb, b, t)
    return out.reshape(b, t, positional_emb.shape[1])


# speedup vs baseline: 1.7115x; 1.7115x over previous
import functools

import jax
import jax.numpy as jnp
from jax.experimental import pallas as pl


@functools.partial(jax.jit, static_argnums=(1, 2))
def _posemb_tc(table, b, t):
    d = table.shape[1]
    bt = 512
    grid = (t // bt,)

    def body(tab_ref, out_ref):
        out_ref[...] = jnp.broadcast_to(tab_ref[...][None], (b, bt, d))

    return pl.pallas_call(
        body,
        grid=grid,
        in_specs=[pl.BlockSpec((bt, d), lambda i: (i, 0))],
        out_specs=pl.BlockSpec((b, bt, d), lambda i: (0, i, 0)),
        out_shape=jax.ShapeDtypeStruct((b, t, d), jnp.float32),
    )(table)


def kernel(x, positional_emb):
    b, t = x.shape
    return _posemb_tc(positional_emb, b, t)
